# baseline (speedup 1.0000x reference)
import functools

import jax
import jax.numpy as jnp
from jax import lax
from jax.experimental import pallas as pl
from jax.experimental.pallas import tpu as pltpu
from jax.experimental.pallas import tpu_sc as plsc

BATCH = 1024
CTX = 200
HD = 64
NUM_CORES = 2
NUM_SUBCORES = 16
NW = NUM_CORES * NUM_SUBCORES
ROWS_PER_W = BATCH // NW
IDX_PER_W = ROWS_PER_W * CTX
C_STEP = 2 * CTX
N_STEP = IDX_PER_W // C_STEP
CHUNKS = (104, 104, 104, 88)

VOCAB = 1000000
PACK_BLK = 10000
UNPACK_BLK = 8

_mesh = plsc.VectorSubcoreMesh(
    core_axis_name="c",
    subcore_axis_name="s",
    num_cores=NUM_CORES,
    num_subcores=NUM_SUBCORES,
)


def _pack_body(t_ref, o_ref):
    v = t_ref[...].reshape(PACK_BLK // 2, 2, HD)
    o_ref[...] = jnp.concatenate([v[:, 0, :], v[:, 1, :]], axis=1)


def _unpack_body(i_ref, o_ref):
    v = i_ref[...]
    a = v[:, :HD]
    b = v[:, HD:]
    o_ref[...] = jnp.stack([a, b], axis=1).reshape(UNPACK_BLK, CTX, HD)


def _emb_body(x_hbm, table2_hbm, pos2_hbm, out_hbm,
              x_v, idx2_v, rows_v, pos_v, gsem, osem):
    wid = lax.axis_index("s") * NUM_CORES + lax.axis_index("c")
    base = wid * IDX_PER_W
    pltpu.sync_copy(x_hbm.at[pl.ds(base, IDX_PER_W)], x_v)
    pltpu.sync_copy(pos2_hbm, pos_v)

    def halve(i, carry):
        idx2_v[pl.ds(i * 16, 16)] = x_v[pl.ds(i * 16, 16)] >> 1
        return carry

    lax.fori_loop(0, IDX_PER_W // 16, halve, 0)

    def start_fetch(s):
        p = s % 2
        cps = []
        o = 0
        for n in CHUNKS:
            cps.append(
                pltpu.async_copy(
                    table2_hbm.at[idx2_v.at[pl.ds(s * C_STEP + o, n)]],
                    rows_v.at[p, pl.ds(o, n)],
                    gsem.at[p],
                )
            )
            o += n
        return cps

    out_cp = [None, None]
    cps_cur = start_fetch(0)
    for s in range(N_STEP):
        p = s % 2
        if s + 1 < N_STEP:
            q = (s + 1) % 2
            if out_cp[q] is not None:
                out_cp[q].wait()
                out_cp[q] = None
            cps_next = start_fetch(s + 1)
        else:
            cps_next = None
        for cp in cps_cur:
            cp.wait()

        def select_add(j, carry):
            pv = plsc.load_gather(
                x_v, [jnp.full((16,), s * C_STEP + j, jnp.int32)]
            )
            m = (pv & 1) > 0
            dbase = (j & 1) * HD
            orow = j // 2
            prow = orow % (CTX // 2)
            for c in range(HD // 16):
                lo = rows_v[p, j, pl.ds(c * 16, 16)]
                hi = rows_v[p, j, pl.ds(HD + c * 16, 16)]
                sel = jnp.where(m, hi, lo)
                rows_v[p, orow, pl.ds(dbase + c * 16, 16)] = (
                    sel + pos_v[prow, pl.ds(dbase + c * 16, 16)]
                )
            return carry

        lax.fori_loop(0, C_STEP, select_add, 0)

        obase = pl.multiple_of(wid * (IDX_PER_W // 2), 8) + s * (C_STEP // 2)
        out_cp[p] = pltpu.async_copy(
            rows_v.at[p, pl.ds(0, C_STEP // 2)],
            out_hbm.at[pl.ds(obase, C_STEP // 2)],
            osem.at[p],
        )
        cps_cur = cps_next

    for cp in out_cp:
        if cp is not None:
            cp.wait()


@functools.partial(jax.jit, static_argnames=())
def _emb_call(x_flat, table, pos_encoding):
    table2 = pl.pallas_call(
        _pack_body,
        grid=(VOCAB // PACK_BLK,),
        in_specs=[pl.BlockSpec((PACK_BLK, HD), lambda k: (k, 0))],
        out_specs=pl.BlockSpec((PACK_BLK // 2, 2 * HD), lambda k: (k, 0)),
        out_shape=jax.ShapeDtypeStruct((VOCAB // 2, 2 * HD), jnp.float32),
    )(table)

    pos2 = pos_encoding.reshape(CTX // 2, 2 * HD)

    packed = pl.kernel(
        _emb_body,
        out_type=jax.ShapeDtypeStruct((BATCH * CTX // 2, 2 * HD), jnp.float32),
        mesh=_mesh,
        scratch_types=[
            pltpu.VMEM((IDX_PER_W,), jnp.int32),
            pltpu.VMEM((IDX_PER_W,), jnp.int32),
            pltpu.VMEM((2, C_STEP, 2 * HD), jnp.float32),
            pltpu.VMEM((CTX // 2, 2 * HD), jnp.float32),
            pltpu.SemaphoreType.DMA((2,)),
            pltpu.SemaphoreType.DMA((2,)),
        ],
        compiler_params=pltpu.CompilerParams(needs_layout_passes=False),
    )(x_flat, table2, pos2)

    out = pl.pallas_call(
        _unpack_body,
        grid=(BATCH // UNPACK_BLK,),
        in_specs=[
            pl.BlockSpec((UNPACK_BLK * CTX // 2, 2 * HD), lambda k: (k, 0))
        ],
        out_specs=pl.BlockSpec((UNPACK_BLK, CTX, HD), lambda k: (k, 0, 0)),
        out_shape=jax.ShapeDtypeStruct((BATCH, CTX, HD), jnp.float32),
    )(packed)
    return out


def kernel(x, table, pos_encoding):
    x_flat = x.reshape(-1).astype(jnp.int32)
    return _emb_call(x_flat, table, pos_encoding)


# speedup vs baseline: 1.1571x; 1.1571x over previous
import functools

import jax
import jax.numpy as jnp
from jax import lax
from jax.experimental import pallas as pl
from jax.experimental.pallas import tpu as pltpu
from jax.experimental.pallas import tpu_sc as plsc

BATCH = 1024
CTX = 200
HD = 64
NUM_CORES = 2
NUM_SUBCORES = 16
NW = NUM_CORES * NUM_SUBCORES
ROWS_PER_W = BATCH // NW
IDX_PER_W = ROWS_PER_W * CTX
C_STEP = 2 * CTX
N_STEP = IDX_PER_W // C_STEP
CHUNKS = (104, 104, 104, 88)

VOCAB = 1000000
POS_PAD = 208
PACK_BLK = 10000
UNPACK_BLK = 8

_mesh = plsc.VectorSubcoreMesh(
    core_axis_name="c",
    subcore_axis_name="s",
    num_cores=NUM_CORES,
    num_subcores=NUM_SUBCORES,
)


def _pack_body(t_ref, o_ref):
    n = t_ref.shape[0]
    v = t_ref[...].reshape(n // 16, 2, 8, HD)
    w = jnp.concatenate([v[:, 0], v[:, 1]], axis=-1)
    o_ref[...] = w.reshape(n // 2, 2 * HD)


def _unpack_body(i_ref, o_ref):
    w = i_ref[...].reshape(UNPACK_BLK * CTX // 16, 8, 2 * HD)
    a = w[:, :, :HD]
    b = w[:, :, HD:]
    v = jnp.stack([a, b], axis=1)
    o_ref[...] = v.reshape(UNPACK_BLK, CTX, HD)


def _emb_body(x_hbm, table2_hbm, pos2_hbm, out_hbm,
              x_v, idx2_v, rows_v, pos_v, gsem, osem):
    wid = lax.axis_index("s") * NUM_CORES + lax.axis_index("c")
    base = wid * IDX_PER_W
    pltpu.sync_copy(x_hbm.at[pl.ds(base, IDX_PER_W)], x_v)
    pltpu.sync_copy(pos2_hbm, pos_v)

    def remap(i, carry):
        xv = x_v[pl.ds(i * 16, 16)]
        idx2_v[pl.ds(i * 16, 16)] = ((xv >> 4) << 3) | (xv & 7)
        return carry

    lax.fori_loop(0, IDX_PER_W // 16, remap, 0)

    def start_fetch(s):
        p = s % 2
        cps = []
        o = 0
        for n in CHUNKS:
            cps.append(
                pltpu.async_copy(
                    table2_hbm.at[idx2_v.at[pl.ds(s * C_STEP + o, n)]],
                    rows_v.at[p, pl.ds(o, n)],
                    gsem.at[p],
                )
            )
            o += n
        return cps

    out_cp = [None, None]
    cps_cur = start_fetch(0)
    for s in range(N_STEP):
        p = s % 2
        if s + 1 < N_STEP:
            q = (s + 1) % 2
            if out_cp[q] is not None:
                out_cp[q].wait()
                out_cp[q] = None
            cps_next = start_fetch(s + 1)
        else:
            cps_next = None
        for cp in cps_cur:
            cp.wait()

        def select_add(j, carry):
            pv = plsc.load_gather(
                x_v, [jnp.full((16,), s * C_STEP + j, jnp.int32)]
            )
            m = (pv & 8) > 0
            orow = ((j >> 4) << 3) | (j & 7)
            dbase = ((j >> 3) & 1) * HD
            tm = lax.rem(j, CTX)
            prow = ((tm >> 4) << 3) | (tm & 7)
            pbase = ((tm >> 3) & 1) * HD
            for c in range(HD // 16):
                lo = rows_v[p, j, pl.ds(c * 16, 16)]
                hi = rows_v[p, j, pl.ds(HD + c * 16, 16)]
                sel = jnp.where(m, hi, lo)
                rows_v[p, orow, pl.ds(dbase + c * 16, 16)] = (
                    sel + pos_v[prow, pl.ds(pbase + c * 16, 16)]
                )
            return carry

        lax.fori_loop(0, C_STEP, select_add, 0)

        obase = pl.multiple_of(wid * (IDX_PER_W // 2), 8) + s * (C_STEP // 2)
        out_cp[p] = pltpu.async_copy(
            rows_v.at[p, pl.ds(0, C_STEP // 2)],
            out_hbm.at[pl.ds(obase, C_STEP // 2)],
            osem.at[p],
        )
        cps_cur = cps_next

    for cp in out_cp:
        if cp is not None:
            cp.wait()


@functools.partial(jax.jit, static_argnames=())
def _emb_call(x_flat, table, pos_encoding):
    table2 = pl.pallas_call(
        _pack_body,
        grid=(VOCAB // PACK_BLK,),
        in_specs=[pl.BlockSpec((PACK_BLK, HD), lambda k: (k, 0))],
        out_specs=pl.BlockSpec((PACK_BLK // 2, 2 * HD), lambda k: (k, 0)),
        out_shape=jax.ShapeDtypeStruct((VOCAB // 2, 2 * HD), jnp.float32),
    )(table)

    pos_pad = jnp.zeros((POS_PAD, HD), jnp.float32).at[:CTX].set(pos_encoding)
    pos2 = pl.pallas_call(
        _pack_body,
        grid=(1,),
        in_specs=[pl.BlockSpec((POS_PAD, HD), lambda k: (k, 0))],
        out_specs=pl.BlockSpec((POS_PAD // 2, 2 * HD), lambda k: (k, 0)),
        out_shape=jax.ShapeDtypeStruct((POS_PAD // 2, 2 * HD), jnp.float32),
    )(pos_pad)

    packed = pl.kernel(
        _emb_body,
        out_type=jax.ShapeDtypeStruct((BATCH * CTX // 2, 2 * HD), jnp.float32),
        mesh=_mesh,
        scratch_types=[
            pltpu.VMEM((IDX_PER_W,), jnp.int32),
            pltpu.VMEM((IDX_PER_W,), jnp.int32),
            pltpu.VMEM((2, C_STEP, 2 * HD), jnp.float32),
            pltpu.VMEM((POS_PAD // 2, 2 * HD), jnp.float32),
            pltpu.SemaphoreType.DMA((2,)),
            pltpu.SemaphoreType.DMA((2,)),
        ],
        compiler_params=pltpu.CompilerParams(needs_layout_passes=False),
    )(x_flat, table2, pos2)

    out = pl.pallas_call(
        _unpack_body,
        grid=(BATCH // UNPACK_BLK,),
        in_specs=[
            pl.BlockSpec((UNPACK_BLK * CTX // 2, 2 * HD), lambda k: (k, 0))
        ],
        out_specs=pl.BlockSpec((UNPACK_BLK, CTX, HD), lambda k: (k, 0, 0)),
        out_shape=jax.ShapeDtypeStruct((BATCH, CTX, HD), jnp.float32),
    )(packed)
    return out


def kernel(x, table, pos_encoding):
    x_flat = x.reshape(-1).astype(jnp.int32)
    return _emb_call(x_flat, table, pos_encoding)


# speedup vs baseline: 1.4014x; 1.2112x over previous
import functools

import jax
import jax.numpy as jnp
from jax import lax
from jax.experimental import pallas as pl
from jax.experimental.pallas import tpu as pltpu
from jax.experimental.pallas import tpu_sc as plsc

BATCH = 1024
CTX = 200
HD = 64
NUM_CORES = 2
NUM_SUBCORES = 16
NW = NUM_CORES * NUM_SUBCORES
ROWS_PER_W = BATCH // NW
IDX_PER_W = ROWS_PER_W * CTX
R_STEP = 2
C_STEP = R_STEP * CTX
N_STEP = ROWS_PER_W // R_STEP
CHUNKS = (104, 104, 104, 88)

_mesh = plsc.VectorSubcoreMesh(
    core_axis_name="c",
    subcore_axis_name="s",
    num_cores=NUM_CORES,
    num_subcores=NUM_SUBCORES,
)


def _emb_body(x_hbm, table_hbm, pos_hbm, out_hbm, idx_v, rows_v, pos_v, gsem, osem):
    wid = lax.axis_index("s") * NUM_CORES + lax.axis_index("c")
    base = wid * IDX_PER_W
    pltpu.sync_copy(x_hbm.at[pl.ds(base, IDX_PER_W)], idx_v)
    pltpu.sync_copy(pos_hbm, pos_v)

    def start_gathers(s):
        p = s % 2
        cps = []
        o = 0
        for n in CHUNKS:
            cps.append(
                pltpu.async_copy(
                    table_hbm.at[idx_v.at[pl.ds(s * C_STEP + o, n)]],
                    rows_v.at[p, pl.ds(o, n)],
                    gsem.at[p],
                )
            )
            o += n
        return cps

    out_cp = [None, None]
    cps_cur = start_gathers(0)
    for s in range(N_STEP):
        p = s % 2
        if s + 1 < N_STEP:
            q = (s + 1) % 2
            if out_cp[q] is not None:
                out_cp[q].wait()
                out_cp[q] = None
            cps_next = start_gathers(s + 1)
        else:
            cps_next = None
        for cp in cps_cur:
            cp.wait()

        for r in range(R_STEP):
            def add_pos(j, carry):
                for c in range(HD // 16):
                    plsc.addupdate(
                        rows_v.at[p, r * CTX + j, pl.ds(c * 16, 16)],
                        pos_v[j, pl.ds(c * 16, 16)],
                    )
                return carry

            lax.fori_loop(0, CTX, add_pos, 0)

        out_cp[p] = pltpu.async_copy(
            rows_v.at[p],
            out_hbm.at[pl.ds(base + s * C_STEP, C_STEP)],
            osem.at[p],
        )
        cps_cur = cps_next

    for cp in out_cp:
        if cp is not None:
            cp.wait()


@functools.partial(jax.jit, static_argnames=())
def _emb_call(x_flat, table, pos_encoding):
    return pl.kernel(
        _emb_body,
        out_type=jax.ShapeDtypeStruct((BATCH * CTX, HD), jnp.float32),
        mesh=_mesh,
        scratch_types=[
            pltpu.VMEM((IDX_PER_W,), jnp.int32),
            pltpu.VMEM((2, C_STEP, HD), jnp.float32),
            pltpu.VMEM((CTX, HD), jnp.float32),
            pltpu.SemaphoreType.DMA((2,)),
            pltpu.SemaphoreType.DMA((2,)),
        ],
        compiler_params=pltpu.CompilerParams(use_tc_tiling_on_sc=False),
    )(x_flat, table, pos_encoding)


def kernel(x, table, pos_encoding):
    x_flat = x.reshape(-1).astype(jnp.int32)
    out = _emb_call(x_flat, table, pos_encoding)
    return out.reshape(BATCH, CTX, HD)


# speedup vs baseline: 1.4017x; 1.0002x over previous
import functools

import jax
import jax.numpy as jnp
from jax import lax
from jax.experimental import pallas as pl
from jax.experimental.pallas import tpu as pltpu
from jax.experimental.pallas import tpu_sc as plsc

BATCH = 1024
CTX = 200
HD = 64
NUM_CORES = 2
NUM_SUBCORES = 16
NW = NUM_CORES * NUM_SUBCORES
ROWS_PER_W = BATCH // NW
IDX_PER_W = ROWS_PER_W * CTX
R_STEP = 4
C_STEP = R_STEP * CTX
N_STEP = ROWS_PER_W // R_STEP
CHUNKS = (112, 112, 112, 112, 112, 112, 112, 16)

_mesh = plsc.VectorSubcoreMesh(
    core_axis_name="c",
    subcore_axis_name="s",
    num_cores=NUM_CORES,
    num_subcores=NUM_SUBCORES,
)


def _emb_body(x_hbm, table_hbm, pos_hbm, out_hbm, idx_v, rows_v, pos_v, gsem, osem):
    wid = lax.axis_index("s") * NUM_CORES + lax.axis_index("c")
    base = wid * IDX_PER_W
    pltpu.sync_copy(x_hbm.at[pl.ds(base, IDX_PER_W)], idx_v)
    pltpu.sync_copy(pos_hbm, pos_v)

    def start_gathers(s):
        p = s % 2
        cps = []
        o = 0
        for n in CHUNKS:
            cps.append(
                pltpu.async_copy(
                    table_hbm.at[idx_v.at[pl.ds(s * C_STEP + o, n)]],
                    rows_v.at[p, pl.ds(o, n)],
                    gsem.at[p],
                )
            )
            o += n
        return cps

    out_cp = [None, None]
    cps_cur = start_gathers(0)
    for s in range(N_STEP):
        p = s % 2
        if s + 1 < N_STEP:
            q = (s + 1) % 2
            if out_cp[q] is not None:
                out_cp[q].wait()
                out_cp[q] = None
            cps_next = start_gathers(s + 1)
        else:
            cps_next = None
        for cp in cps_cur:
            cp.wait()

        for r in range(R_STEP):
            def add_pos(j, carry):
                for c in range(HD // 16):
                    plsc.addupdate(
                        rows_v.at[p, r * CTX + j, pl.ds(c * 16, 16)],
                        pos_v[j, pl.ds(c * 16, 16)],
                    )
                return carry

            lax.fori_loop(0, CTX, add_pos, 0)

        out_cp[p] = pltpu.async_copy(
            rows_v.at[p],
            out_hbm.at[pl.ds(base + s * C_STEP, C_STEP)],
            osem.at[p],
        )
        cps_cur = cps_next

    for cp in out_cp:
        if cp is not None:
            cp.wait()


@functools.partial(jax.jit, static_argnames=())
def _emb_call(x_flat, table, pos_encoding):
    return pl.kernel(
        _emb_body,
        out_type=jax.ShapeDtypeStruct((BATCH * CTX, HD), jnp.float32),
        mesh=_mesh,
        scratch_types=[
            pltpu.VMEM((IDX_PER_W,), jnp.int32),
            pltpu.VMEM((2, C_STEP, HD), jnp.float32),
            pltpu.VMEM((CTX, HD), jnp.float32),
            pltpu.SemaphoreType.DMA((2,)),
            pltpu.SemaphoreType.DMA((2,)),
        ],
        compiler_params=pltpu.CompilerParams(use_tc_tiling_on_sc=False),
    )(x_flat, table, pos_encoding)


def kernel(x, table, pos_encoding):
    x_flat = x.reshape(-1).astype(jnp.int32)
    out = _emb_call(x_flat, table, pos_encoding)
    return out.reshape(BATCH, CTX, HD)
